# Initial kernel scaffold; baseline (speedup 1.0000x reference)
import jax
import jax.numpy as jnp
from jax.experimental import pallas as pl


def kernel(x, edge_index, edge_attr, batch, c1_W1, c1_b1, c1_W2, c1_b2, c1_root, c1_bias, c2_W1, c2_b1, c2_W2, c2_b2, c2_root, c2_bias, fc1_W, fc1_b, out_W, out_b):
    raise NotImplementedError("write your pallas kernel here")


# speedup vs baseline: 1.2886x; 1.2886x over previous
import functools

import jax
import jax.numpy as jnp
from jax import lax
from jax.experimental import pallas as pl
from jax.experimental.pallas import tpu as pltpu
from jax.experimental.pallas import tpu_sc as plsc

_NUM_GRAPHS = 128
_SC_CORES = 2
_SC_SUBCORES = 16
_SC_WORKERS = _SC_CORES * _SC_SUBCORES
_K = 80
_J = 10
_CHUNK = _K * _J
_SJ = 2
_SCHUNK = _K * _SJ


def _sc_gather(table, idx):
    e = idx.shape[0]
    per_w = e // _SC_WORKERS
    mesh = plsc.VectorSubcoreMesh(core_axis_name="c", subcore_axis_name="s")

    @functools.partial(
        pl.kernel,
        mesh=mesh,
        out_type=jax.ShapeDtypeStruct((e, 128), jnp.float32),
        scratch_types=[
            pltpu.VMEM((_CHUNK,), jnp.int32),
            pltpu.VMEM((_CHUNK, 128), jnp.float32),
            pltpu.SemaphoreType.DMA,
        ],
    )
    def k(table_hbm, idx_hbm, out_hbm, idx_v, rows_v, sem):
        wid = lax.axis_index("s") * _SC_CORES + lax.axis_index("c")
        base = wid * per_w

        @pl.loop(0, per_w // _CHUNK)
        def _(t):
            pltpu.sync_copy(idx_hbm.at[pl.ds(base + t * _CHUNK, _CHUNK)],
                            idx_v)
            handles = [
                pltpu.async_copy(table_hbm.at[idx_v.at[pl.ds(j * _K, _K)]],
                                 rows_v.at[pl.ds(j * _K, _K)], sem)
                for j in range(_J)
            ]
            for h in handles:
                h.wait()
            pltpu.sync_copy(rows_v,
                            out_hbm.at[pl.ds(base + t * _CHUNK, _CHUNK)])

    return k(table, idx)


def _sc_scatter_add(msg, dst, zeros):
    e, _ = msg.shape
    n_pad = zeros.shape[0]
    per_w = e // _SC_WORKERS
    rows_per_sub = n_pad // _SC_SUBCORES
    mesh = plsc.VectorSubcoreMesh(core_axis_name="c", subcore_axis_name="s")

    @functools.partial(
        pl.kernel,
        mesh=mesh,
        out_type=jax.ShapeDtypeStruct((2 * n_pad, 128), jnp.float32),
        scratch_types=[
            pltpu.VMEM((_SJ, _K), jnp.int32),
            pltpu.VMEM((_SCHUNK, 128), jnp.float32),
            pltpu.VMEM_SHARED((n_pad, 128), jnp.float32),
            pltpu.SemaphoreType.DMA,
            pltpu.SemaphoreType.DMA,
        ],
    )
    def k(msg_hbm, dst_hbm, zero_hbm, out_hbm, idx_v, rows_v, acc_sh, sem,
          isem):
        cid = lax.axis_index("c")
        sid = lax.axis_index("s")
        wid = sid * _SC_CORES + cid
        base = wid * per_w
        my_rows = sid * rows_per_sub

        pltpu.sync_copy(zero_hbm.at[pl.ds(my_rows, rows_per_sub)],
                        acc_sh.at[pl.ds(my_rows, rows_per_sub)])
        plsc.subcore_barrier()

        @pl.loop(0, per_w // _SCHUNK)
        def _(t):
            ih = [
                pltpu.async_copy(
                    dst_hbm.at[pl.ds(base + t * _SCHUNK + j * _K, _K)],
                    idx_v.at[j], isem)
                for j in range(_SJ)
            ]
            pltpu.sync_copy(msg_hbm.at[pl.ds(base + t * _SCHUNK, _SCHUNK)],
                            rows_v)
            for h in ih:
                h.wait()
            for j in range(_SJ):
                pltpu.sync_copy(rows_v.at[pl.ds(j * _K, _K)],
                                acc_sh.at[idx_v.at[j]], add=True)

        plsc.subcore_barrier()
        pltpu.sync_copy(
            acc_sh.at[pl.ds(my_rows, rows_per_sub)],
            out_hbm.at[pl.ds(cid * n_pad + my_rows, rows_per_sub)])

    return k(msg, dst, zeros)


def _edge_body(in_c, out_c, blk, ea_ref, xj_ref, w1_ref, b1_ref, w2_ref,
               b2_ref, out_ref):
    h = jnp.maximum(
        jnp.dot(ea_ref[...], w1_ref[...],
                preferred_element_type=jnp.float32) + b1_ref[...], 0.0)
    g = jnp.dot(h, w2_ref[...],
                preferred_element_type=jnp.float32) + b2_ref[...]
    xj = xj_ref[...]
    acc = xj[:, 0:1] * g[:, 0:out_c]
    for i in range(1, in_c):
        acc = acc + xj[:, i:i + 1] * g[:, i * out_c:(i + 1) * out_c]
    out_ref[...] = jnp.concatenate(
        [acc, jnp.zeros((blk, 128 - out_c), jnp.float32)], axis=1)


def _edge_messages(ea, xj, w1, b1, w2, b2, in_c, out_c, blk):
    e = ea.shape[0]
    hid = w1.shape[1]
    kfn = functools.partial(_edge_body, in_c, out_c, blk)
    return pl.pallas_call(
        kfn,
        grid=(e // blk,),
        in_specs=[
            pl.BlockSpec((blk, ea.shape[1]), lambda i: (i, 0)),
            pl.BlockSpec((blk, 128), lambda i: (i, 0)),
            pl.BlockSpec(w1.shape, lambda i: (0, 0)),
            pl.BlockSpec((1, hid), lambda i: (0, 0)),
            pl.BlockSpec(w2.shape, lambda i: (0, 0)),
            pl.BlockSpec((1, in_c * out_c), lambda i: (0, 0)),
        ],
        out_specs=pl.BlockSpec((blk, 128), lambda i: (i, 0)),
        out_shape=jax.ShapeDtypeStruct((e, 128), jnp.float32),
    )(ea, xj, w1, b1[None, :], w2, b2[None, :])


def _node1_body(n, n_pad, p_ref, x_ref, r_ref, b_ref, o_ref):
    agg = p_ref[0:n, 0:32] + p_ref[n_pad:n_pad + n, 0:32]
    h = jnp.maximum(
        agg + jnp.dot(x_ref[...], r_ref[...],
                      preferred_element_type=jnp.float32) + b_ref[...], 0.0)
    o_ref[...] = jnp.concatenate(
        [h, jnp.zeros((n, 96), jnp.float32)], axis=1)


def _node1(parts, x, root, bias):
    n = x.shape[0]
    return pl.pallas_call(
        functools.partial(_node1_body, n, parts.shape[0] // 2),
        out_shape=jax.ShapeDtypeStruct((n, 128), jnp.float32),
    )(parts, x, root, bias[None, :])


def _head_body(n, n_pad, p_ref, h_ref, r_ref, b_ref, batch_ref, fw_ref,
               fb_ref, ow_ref, ob_ref, o_ref):
    agg = p_ref[0:n, 0:16] + p_ref[n_pad:n_pad + n, 0:16]
    h2 = jnp.maximum(
        agg + jnp.dot(h_ref[:, 0:32], r_ref[...],
                      preferred_element_type=jnp.float32) + b_ref[...], 0.0)
    seg = lax.broadcasted_iota(jnp.int32, (_NUM_GRAPHS, n), 0)
    onehot = jnp.where(seg == batch_ref[...], 1.0, 0.0)
    g = jnp.dot(onehot, h2, preferred_element_type=jnp.float32)
    g = jnp.maximum(
        jnp.dot(g, fw_ref[...], preferred_element_type=jnp.float32)
        + fb_ref[...], 0.0)
    o_ref[...] = jnp.dot(g, ow_ref[...],
                         preferred_element_type=jnp.float32) + ob_ref[...]


def _head(parts2, h, root2, bias2, batch, fc1_w, fc1_b, out_w, out_b):
    n = h.shape[0]
    return pl.pallas_call(
        functools.partial(_head_body, n, parts2.shape[0] // 2),
        out_shape=jax.ShapeDtypeStruct((_NUM_GRAPHS, 1), jnp.float32),
    )(parts2, h, root2, bias2[None, :], batch[None, :], fc1_w, fc1_b[None, :],
      out_w, out_b[None, :])


def kernel(x, edge_index, edge_attr, batch, c1_W1, c1_b1, c1_W2, c1_b2,
           c1_root, c1_bias, c2_W1, c2_b1, c2_W2, c2_b2, c2_root, c2_bias,
           fc1_W, fc1_b, out_W, out_b):
    n = x.shape[0]
    n_pad = -(-n // (8 * _SC_SUBCORES)) * (8 * _SC_SUBCORES)
    src = edge_index[0]
    dst = edge_index[1]

    x128 = jnp.pad(x, ((0, 0), (0, 128 - x.shape[1])))
    zeros = jnp.zeros((n_pad, 128), jnp.float32)

    xj = _sc_gather(x128, src)
    msg1 = _edge_messages(edge_attr, xj, c1_W1, c1_b1, c1_W2, c1_b2,
                          in_c=11, out_c=32, blk=2000)
    parts1 = _sc_scatter_add(msg1, dst, zeros)
    h = _node1(parts1, x, c1_root, c1_bias)

    hj = _sc_gather(h, src)
    msg2 = _edge_messages(edge_attr, hj, c2_W1, c2_b1, c2_W2, c2_b2,
                          in_c=32, out_c=16, blk=2000)
    parts2 = _sc_scatter_add(msg2, dst, zeros)
    return _head(parts2, h, c2_root, c2_bias, batch, fc1_W, fc1_b,
                 out_W, out_b)


# speedup vs baseline: 4.3828x; 3.4012x over previous
import functools

import jax
import jax.numpy as jnp
from jax import lax
from jax.experimental import pallas as pl
from jax.experimental.pallas import tpu as pltpu
from jax.experimental.pallas import tpu_sc as plsc

_NUM_GRAPHS = 128
_SC_CORES = 2
_SC_SUBCORES = 16
_SC_WORKERS = _SC_CORES * _SC_SUBCORES
_K = 80
_J = 10
_CHUNK = _K * _J
_SJ = 2
_SCHUNK = _K * _SJ


def _sc_gather(table, idx):
    e = idx.shape[0]
    per_w = e // _SC_WORKERS
    mesh = plsc.VectorSubcoreMesh(core_axis_name="c", subcore_axis_name="s")

    @functools.partial(
        pl.kernel,
        mesh=mesh,
        out_type=jax.ShapeDtypeStruct((e, 128), jnp.float32),
        scratch_types=[
            pltpu.VMEM((_CHUNK,), jnp.int32),
            pltpu.VMEM((_CHUNK, 128), jnp.float32),
            pltpu.SemaphoreType.DMA,
        ],
    )
    def k(table_hbm, idx_hbm, out_hbm, idx_v, rows_v, sem):
        wid = lax.axis_index("s") * _SC_CORES + lax.axis_index("c")
        base = wid * per_w

        @pl.loop(0, per_w // _CHUNK)
        def _(t):
            pltpu.sync_copy(idx_hbm.at[pl.ds(base + t * _CHUNK, _CHUNK)],
                            idx_v)
            handles = [
                pltpu.async_copy(table_hbm.at[idx_v.at[pl.ds(j * _K, _K)]],
                                 rows_v.at[pl.ds(j * _K, _K)], sem)
                for j in range(_J)
            ]
            for h in handles:
                h.wait()
            pltpu.sync_copy(rows_v,
                            out_hbm.at[pl.ds(base + t * _CHUNK, _CHUNK)])

    return k(table, idx)


def _sc_scatter_add(msg, dst, zeros):
    e, _ = msg.shape
    n_pad = zeros.shape[0]
    per_w = e // _SC_WORKERS
    rows_per_sub = n_pad // _SC_SUBCORES
    mesh = plsc.VectorSubcoreMesh(core_axis_name="c", subcore_axis_name="s")

    @functools.partial(
        pl.kernel,
        mesh=mesh,
        out_type=jax.ShapeDtypeStruct((2 * n_pad, 128), jnp.float32),
        scratch_types=[
            pltpu.VMEM((_SJ, _K), jnp.int32),
            pltpu.VMEM((_SCHUNK, 128), jnp.float32),
            pltpu.VMEM_SHARED((n_pad, 128), jnp.float32),
            pltpu.SemaphoreType.DMA,
            pltpu.SemaphoreType.DMA,
        ],
    )
    def k(msg_hbm, dst_hbm, zero_hbm, out_hbm, idx_v, rows_v, acc_sh, sem,
          isem):
        cid = lax.axis_index("c")
        sid = lax.axis_index("s")
        wid = sid * _SC_CORES + cid
        base = wid * per_w
        my_rows = sid * rows_per_sub

        pltpu.sync_copy(zero_hbm.at[pl.ds(my_rows, rows_per_sub)],
                        acc_sh.at[pl.ds(my_rows, rows_per_sub)])
        plsc.subcore_barrier()

        @pl.loop(0, per_w // _SCHUNK)
        def _(t):
            ih = [
                pltpu.async_copy(
                    dst_hbm.at[pl.ds(base + t * _SCHUNK + j * _K, _K)],
                    idx_v.at[j], isem)
                for j in range(_SJ)
            ]
            pltpu.sync_copy(msg_hbm.at[pl.ds(base + t * _SCHUNK, _SCHUNK)],
                            rows_v)
            for h in ih:
                h.wait()
            for j in range(_SJ):
                pltpu.sync_copy(rows_v.at[pl.ds(j * _K, _K)],
                                acc_sh.at[idx_v.at[j]], add=True)

        plsc.subcore_barrier()
        pltpu.sync_copy(
            acc_sh.at[pl.ds(my_rows, rows_per_sub)],
            out_hbm.at[pl.ds(cid * n_pad + my_rows, rows_per_sub)])

    return k(msg, dst, zeros)


def _edge_body(in_c, out_c, blk, ea_ref, xj_ref, w1_ref, b1_ref, w2_ref,
               b2_ref, rep_ref, sum_ref, out_ref):
    h = jnp.maximum(
        jnp.dot(ea_ref[...], w1_ref[...],
                preferred_element_type=jnp.float32) + b1_ref[...], 0.0)
    g = jnp.dot(h, w2_ref[...],
                preferred_element_type=jnp.float32) + b2_ref[...]
    xjr = jnp.dot(xj_ref[...], rep_ref[...],
                  preferred_element_type=jnp.float32)
    msg = jnp.dot(xjr * g, sum_ref[...], preferred_element_type=jnp.float32)
    out_ref[...] = jnp.concatenate(
        [msg, jnp.zeros((blk, 128 - out_c), jnp.float32)], axis=1)


def _edge_messages(ea, xj, w1, b1, w2, b2, in_c, out_c, blk):
    e = ea.shape[0]
    hid = w1.shape[1]
    ic_oc = in_c * out_c
    rep = (jnp.arange(ic_oc)[None, :] // out_c
           == jnp.arange(128)[:, None]).astype(jnp.float32)
    summ = (jnp.arange(ic_oc)[:, None] % out_c
            == jnp.arange(out_c)[None, :]).astype(jnp.float32)
    kfn = functools.partial(_edge_body, in_c, out_c, blk)
    return pl.pallas_call(
        kfn,
        grid=(e // blk,),
        in_specs=[
            pl.BlockSpec((blk, ea.shape[1]), lambda i: (i, 0)),
            pl.BlockSpec((blk, 128), lambda i: (i, 0)),
            pl.BlockSpec(w1.shape, lambda i: (0, 0)),
            pl.BlockSpec((1, hid), lambda i: (0, 0)),
            pl.BlockSpec(w2.shape, lambda i: (0, 0)),
            pl.BlockSpec((1, ic_oc), lambda i: (0, 0)),
            pl.BlockSpec((128, ic_oc), lambda i: (0, 0)),
            pl.BlockSpec((ic_oc, out_c), lambda i: (0, 0)),
        ],
        out_specs=pl.BlockSpec((blk, 128), lambda i: (i, 0)),
        out_shape=jax.ShapeDtypeStruct((e, 128), jnp.float32),
    )(ea, xj, w1, b1[None, :], w2, b2[None, :], rep, summ)


def _node1_body(n, n_pad, p_ref, x_ref, r_ref, b_ref, o_ref):
    agg = p_ref[0:n, 0:32] + p_ref[n_pad:n_pad + n, 0:32]
    h = jnp.maximum(
        agg + jnp.dot(x_ref[...], r_ref[...],
                      preferred_element_type=jnp.float32) + b_ref[...], 0.0)
    o_ref[...] = jnp.concatenate(
        [h, jnp.zeros((n, 96), jnp.float32)], axis=1)


def _node1(parts, x, root, bias):
    n = x.shape[0]
    return pl.pallas_call(
        functools.partial(_node1_body, n, parts.shape[0] // 2),
        out_shape=jax.ShapeDtypeStruct((n, 128), jnp.float32),
    )(parts, x, root, bias[None, :])


def _head_body(n, n_pad, p_ref, h_ref, r_ref, b_ref, batch_ref, fw_ref,
               fb_ref, ow_ref, ob_ref, o_ref):
    agg = p_ref[0:n, 0:16] + p_ref[n_pad:n_pad + n, 0:16]
    h2 = jnp.maximum(
        agg + jnp.dot(h_ref[:, 0:32], r_ref[...],
                      preferred_element_type=jnp.float32) + b_ref[...], 0.0)
    seg = lax.broadcasted_iota(jnp.int32, (_NUM_GRAPHS, n), 0)
    onehot = jnp.where(seg == batch_ref[...], 1.0, 0.0)
    g = jnp.dot(onehot, h2, preferred_element_type=jnp.float32)
    g = jnp.maximum(
        jnp.dot(g, fw_ref[...], preferred_element_type=jnp.float32)
        + fb_ref[...], 0.0)
    o_ref[...] = jnp.dot(g, ow_ref[...],
                         preferred_element_type=jnp.float32) + ob_ref[...]


def _head(parts2, h, root2, bias2, batch, fc1_w, fc1_b, out_w, out_b):
    n = h.shape[0]
    return pl.pallas_call(
        functools.partial(_head_body, n, parts2.shape[0] // 2),
        out_shape=jax.ShapeDtypeStruct((_NUM_GRAPHS, 1), jnp.float32),
    )(parts2, h, root2, bias2[None, :], batch[None, :], fc1_w, fc1_b[None, :],
      out_w, out_b[None, :])


def kernel(x, edge_index, edge_attr, batch, c1_W1, c1_b1, c1_W2, c1_b2,
           c1_root, c1_bias, c2_W1, c2_b1, c2_W2, c2_b2, c2_root, c2_bias,
           fc1_W, fc1_b, out_W, out_b):
    n = x.shape[0]
    n_pad = -(-n // (8 * _SC_SUBCORES)) * (8 * _SC_SUBCORES)
    src = edge_index[0]
    dst = edge_index[1]

    x128 = jnp.pad(x, ((0, 0), (0, 128 - x.shape[1])))
    zeros = jnp.zeros((n_pad, 128), jnp.float32)

    xj = _sc_gather(x128, src)
    msg1 = _edge_messages(edge_attr, xj, c1_W1, c1_b1, c1_W2, c1_b2,
                          in_c=11, out_c=32, blk=2000)
    parts1 = _sc_scatter_add(msg1, dst, zeros)
    h = _node1(parts1, x, c1_root, c1_bias)

    hj = _sc_gather(h, src)
    msg2 = _edge_messages(edge_attr, hj, c2_W1, c2_b1, c2_W2, c2_b2,
                          in_c=32, out_c=16, blk=2000)
    parts2 = _sc_scatter_add(msg2, dst, zeros)
    return _head(parts2, h, c2_root, c2_bias, batch, fc1_W, fc1_b,
                 out_W, out_b)


# speedup vs baseline: 5.0583x; 1.1541x over previous
import functools

import jax
import jax.numpy as jnp
from jax import lax
from jax.experimental import pallas as pl
from jax.experimental.pallas import tpu as pltpu
from jax.experimental.pallas import tpu_sc as plsc

_NUM_GRAPHS = 128
_SC_CORES = 2
_SC_SUBCORES = 16
_SC_WORKERS = _SC_CORES * _SC_SUBCORES
_K = 80
_J = 5
_CHUNK = _K * _J
_SK = 40
_SJ = 5
_SCHUNK = _SK * _SJ


def _sc_gather(table, idx, e_off, e_len):
    e = e_len
    per_w = e // _SC_WORKERS
    mesh = plsc.VectorSubcoreMesh(core_axis_name="c", subcore_axis_name="s")

    @functools.partial(
        pl.kernel,
        mesh=mesh,
        out_type=jax.ShapeDtypeStruct((e, 128), jnp.float32),
        scratch_types=[
            pltpu.VMEM((_CHUNK,), jnp.int32),
            pltpu.VMEM((_CHUNK, 128), jnp.float32),
            pltpu.SemaphoreType.DMA,
        ],
    )
    def k(table_hbm, idx_hbm, out_hbm, idx_v, rows_v, sem):
        wid = lax.axis_index("s") * _SC_CORES + lax.axis_index("c")
        base = wid * per_w

        @pl.loop(0, per_w // _CHUNK)
        def _(t):
            pltpu.sync_copy(
                idx_hbm.at[pl.ds(e_off + base + t * _CHUNK, _CHUNK)],
                idx_v)
            handles = [
                pltpu.async_copy(table_hbm.at[idx_v.at[pl.ds(j * _K, _K)]],
                                 rows_v.at[pl.ds(j * _K, _K)], sem)
                for j in range(_J)
            ]
            for h in handles:
                h.wait()
            pltpu.sync_copy(rows_v,
                            out_hbm.at[pl.ds(base + t * _CHUNK, _CHUNK)])

    return k(table, idx)


def _sc_scatter_add(msg, dst, e_off, zeros):
    e, _ = msg.shape
    n_pad = zeros.shape[0]
    per_w = e // _SC_WORKERS
    rows_per_sub = n_pad // _SC_SUBCORES
    mesh = plsc.VectorSubcoreMesh(core_axis_name="c", subcore_axis_name="s")

    @functools.partial(
        pl.kernel,
        mesh=mesh,
        out_type=jax.ShapeDtypeStruct((2 * n_pad, 128), jnp.float32),
        scratch_types=[
            pltpu.VMEM((_SJ, _SK), jnp.int32),
            pltpu.VMEM((_SCHUNK, 128), jnp.float32),
            pltpu.VMEM_SHARED((n_pad, 128), jnp.float32),
            pltpu.SemaphoreType.DMA,
            pltpu.SemaphoreType.DMA,
        ],
    )
    def k(msg_hbm, dst_hbm, zero_hbm, out_hbm, idx_v, rows_v, acc_sh, sem,
          isem):
        cid = lax.axis_index("c")
        sid = lax.axis_index("s")
        wid = sid * _SC_CORES + cid
        base = wid * per_w
        my_rows = sid * rows_per_sub

        pltpu.sync_copy(zero_hbm.at[pl.ds(my_rows, rows_per_sub)],
                        acc_sh.at[pl.ds(my_rows, rows_per_sub)])
        plsc.subcore_barrier()

        @pl.loop(0, per_w // _SCHUNK)
        def _(t):
            ih = [
                pltpu.async_copy(
                    dst_hbm.at[
                        pl.ds(e_off + base + t * _SCHUNK + j * _SK, _SK)],
                    idx_v.at[j], isem)
                for j in range(_SJ)
            ]
            pltpu.sync_copy(msg_hbm.at[pl.ds(base + t * _SCHUNK, _SCHUNK)],
                            rows_v)
            for h in ih:
                h.wait()
            for j in range(_SJ):
                pltpu.sync_copy(rows_v.at[pl.ds(j * _SK, _SK)],
                                acc_sh.at[idx_v.at[j]], add=True)

        plsc.subcore_barrier()
        pltpu.sync_copy(
            acc_sh.at[pl.ds(my_rows, rows_per_sub)],
            out_hbm.at[pl.ds(cid * n_pad + my_rows, rows_per_sub)])

    return k(msg, dst, zeros)


def _edge_body(in_c, out_c, blk, ea_ref, xj_ref, w1_ref, b1_ref, w2_ref,
               b2_ref, rep_ref, sum_ref, out_ref):
    h = jnp.maximum(
        jnp.dot(ea_ref[...], w1_ref[...],
                preferred_element_type=jnp.float32) + b1_ref[...], 0.0)
    g = jnp.dot(h, w2_ref[...],
                preferred_element_type=jnp.float32) + b2_ref[...]
    xjr = jnp.dot(xj_ref[...], rep_ref[...],
                  preferred_element_type=jnp.float32)
    msg = jnp.dot(xjr * g, sum_ref[...], preferred_element_type=jnp.float32)
    out_ref[...] = jnp.concatenate(
        [msg, jnp.zeros((blk, 128 - out_c), jnp.float32)], axis=1)


def _edge_messages(ea, xj, w1, b1, w2, b2, in_c, out_c, blk, blk_off):
    e = xj.shape[0]
    hid = w1.shape[1]
    ic_oc = in_c * out_c
    rep = (jnp.arange(ic_oc)[None, :] // out_c
           == jnp.arange(128)[:, None]).astype(jnp.float32)
    summ = (jnp.arange(ic_oc)[:, None] % out_c
            == jnp.arange(out_c)[None, :]).astype(jnp.float32)
    kfn = functools.partial(_edge_body, in_c, out_c, blk)
    return pl.pallas_call(
        kfn,
        grid=(e // blk,),
        in_specs=[
            pl.BlockSpec((blk, ea.shape[1]), lambda i: (i + blk_off, 0)),
            pl.BlockSpec((blk, 128), lambda i: (i, 0)),
            pl.BlockSpec(w1.shape, lambda i: (0, 0)),
            pl.BlockSpec((1, hid), lambda i: (0, 0)),
            pl.BlockSpec(w2.shape, lambda i: (0, 0)),
            pl.BlockSpec((1, ic_oc), lambda i: (0, 0)),
            pl.BlockSpec((128, ic_oc), lambda i: (0, 0)),
            pl.BlockSpec((ic_oc, out_c), lambda i: (0, 0)),
        ],
        out_specs=pl.BlockSpec((blk, 128), lambda i: (i, 0)),
        out_shape=jax.ShapeDtypeStruct((e, 128), jnp.float32),
    )(ea, xj, w1, b1[None, :], w2, b2[None, :], rep, summ)


def _node1_body(n, n_pad, pa_ref, pb_ref, x_ref, r_ref, b_ref, o_ref):
    agg = (pa_ref[0:n, 0:32] + pa_ref[n_pad:n_pad + n, 0:32]
           + pb_ref[0:n, 0:32] + pb_ref[n_pad:n_pad + n, 0:32])
    h = jnp.maximum(
        agg + jnp.dot(x_ref[...], r_ref[...],
                      preferred_element_type=jnp.float32) + b_ref[...], 0.0)
    o_ref[...] = jnp.concatenate(
        [h, jnp.zeros((n, 96), jnp.float32)], axis=1)


def _node1(parts_a, parts_b, x, root, bias):
    n = x.shape[0]
    return pl.pallas_call(
        functools.partial(_node1_body, n, parts_a.shape[0] // 2),
        out_shape=jax.ShapeDtypeStruct((n, 128), jnp.float32),
    )(parts_a, parts_b, x, root, bias[None, :])


def _head_body(n, n_pad, pa_ref, pb_ref, h_ref, r_ref, b_ref, batch_ref,
               fw_ref, fb_ref, ow_ref, ob_ref, o_ref):
    agg = (pa_ref[0:n, 0:16] + pa_ref[n_pad:n_pad + n, 0:16]
           + pb_ref[0:n, 0:16] + pb_ref[n_pad:n_pad + n, 0:16])
    h2 = jnp.maximum(
        agg + jnp.dot(h_ref[:, 0:32], r_ref[...],
                      preferred_element_type=jnp.float32) + b_ref[...], 0.0)
    seg = lax.broadcasted_iota(jnp.int32, (_NUM_GRAPHS, n), 0)
    onehot = jnp.where(seg == batch_ref[...], 1.0, 0.0)
    g = jnp.dot(onehot, h2, preferred_element_type=jnp.float32)
    g = jnp.maximum(
        jnp.dot(g, fw_ref[...], preferred_element_type=jnp.float32)
        + fb_ref[...], 0.0)
    o_ref[...] = jnp.dot(g, ow_ref[...],
                         preferred_element_type=jnp.float32) + ob_ref[...]


def _head(parts2_a, parts2_b, h, root2, bias2, batch, fc1_w, fc1_b, out_w,
          out_b):
    n = h.shape[0]
    return pl.pallas_call(
        functools.partial(_head_body, n, parts2_a.shape[0] // 2),
        out_shape=jax.ShapeDtypeStruct((_NUM_GRAPHS, 1), jnp.float32),
    )(parts2_a, parts2_b, h, root2, bias2[None, :], batch[None, :], fc1_w,
      fc1_b[None, :], out_w, out_b[None, :])


def kernel(x, edge_index, edge_attr, batch, c1_W1, c1_b1, c1_W2, c1_b2,
           c1_root, c1_bias, c2_W1, c2_b1, c2_W2, c2_b2, c2_root, c2_bias,
           fc1_W, fc1_b, out_W, out_b):
    n = x.shape[0]
    e = edge_index.shape[1]
    eh = e // 2
    blk = 2000
    n_pad = -(-n // (8 * _SC_SUBCORES)) * (8 * _SC_SUBCORES)
    src = edge_index[0]
    dst = edge_index[1]

    x128 = jnp.pad(x, ((0, 0), (0, 128 - x.shape[1])))
    zeros = jnp.zeros((n_pad, 128), jnp.float32)

    p1 = []
    for s in range(2):
        xj = _sc_gather(x128, src, s * eh, eh)
        msg1 = _edge_messages(edge_attr, xj, c1_W1, c1_b1, c1_W2, c1_b2,
                              in_c=11, out_c=32, blk=blk,
                              blk_off=s * eh // blk)
        p1.append(_sc_scatter_add(msg1, dst, s * eh, zeros))
    h = _node1(p1[0], p1[1], x, c1_root, c1_bias)

    p2 = []
    for s in range(2):
        hj = _sc_gather(h, src, s * eh, eh)
        msg2 = _edge_messages(edge_attr, hj, c2_W1, c2_b1, c2_W2, c2_b2,
                              in_c=32, out_c=16, blk=blk,
                              blk_off=s * eh // blk)
        p2.append(_sc_scatter_add(msg2, dst, s * eh, zeros))
    return _head(p2[0], p2[1], h, c2_root, c2_bias, batch, fc1_W, fc1_b,
                 out_W, out_b)
